# Initial kernel scaffold; baseline (speedup 1.0000x reference)
import jax
import jax.numpy as jnp
from jax.experimental import pallas as pl


def kernel(x, edge_index, W, att_src, att_dst, bias):
    raise NotImplementedError("write your pallas kernel here")


# speedup vs baseline: 35.7220x; 35.7220x over previous
import dataclasses
import functools

import jax
import jax.numpy as jnp
from jax import lax
from jax.experimental import pallas as pl
from jax.experimental.pallas import tpu as pltpu
from jax.experimental.pallas import tpu_sc as plsc

N = 10000
E = 320000
F_IN = 128
H = 8
C = 16
HC = H * C
NEG_SLOPE = 0.2

NUM_SC = 2
NUM_SUB = 16
NW = NUM_SC * NUM_SUB
EDGES_PER_W = E // NW
CHUNK = 80
NCHUNK = EDGES_PER_W // CHUNK
NP = 10240
ROWS_PER_SUB = NP // NUM_SUB


def _proj_body(x_ref, w_ref, ab_ref, xw_ref, as_ref, ad_ref):
    xw = jnp.dot(x_ref[...], w_ref[...], preferred_element_type=jnp.float32)
    xw_ref[...] = xw
    ab = jnp.dot(xw, ab_ref[...], preferred_element_type=jnp.float32)
    as_ref[...] = ab[:, :16]
    ad_ref[...] = ab[:, 16:]


def _project(x, W, AB):
    blk = 400
    grid = (N // blk,)
    return pl.pallas_call(
        _proj_body,
        grid=grid,
        in_specs=[
            pl.BlockSpec((blk, F_IN), lambda i: (i, 0)),
            pl.BlockSpec((F_IN, HC), lambda i: (0, 0)),
            pl.BlockSpec((F_IN, 32), lambda i: (0, 0)),
        ],
        out_specs=[
            pl.BlockSpec((blk, HC), lambda i: (i, 0)),
            pl.BlockSpec((blk, 16), lambda i: (i, 0)),
            pl.BlockSpec((blk, 16), lambda i: (i, 0)),
        ],
        out_shape=[
            jax.ShapeDtypeStruct((N, HC), jnp.float32),
            jax.ShapeDtypeStruct((N, 16), jnp.float32),
            jax.ShapeDtypeStruct((N, 16), jnp.float32),
        ],
    )(x, W, AB)


def _sc_edge_kernel(src, dst, asrc, adst, xw, zeros128, zeros16):
    mesh = plsc.VectorSubcoreMesh(core_axis_name="c", subcore_axis_name="s")
    cp = pltpu.CompilerParams()
    if "needs_layout_passes" in pltpu.CompilerParams.__dataclass_fields__:
        cp = dataclasses.replace(cp, needs_layout_passes=False)
    if "use_tc_tiling_on_sc" in pltpu.CompilerParams.__dataclass_fields__:
        cp = dataclasses.replace(cp, use_tc_tiling_on_sc=False)

    @functools.partial(
        pl.kernel,
        mesh=mesh,
        compiler_params=cp,
        out_type=[
            jax.ShapeDtypeStruct((NUM_SC, NP, HC), jnp.float32),
            jax.ShapeDtypeStruct((NUM_SC, NP, 16), jnp.float32),
        ],
        scratch_types=[
            pltpu.VMEM((1, CHUNK), jnp.int32),
            pltpu.VMEM((1, CHUNK), jnp.int32),
            pltpu.VMEM((CHUNK, 16), jnp.float32),
            pltpu.VMEM((CHUNK, 16), jnp.float32),
            pltpu.VMEM((CHUNK, 16), jnp.float32),
            pltpu.VMEM((CHUNK, HC), jnp.float32),
            pltpu.VMEM((CHUNK, HC), jnp.float32),
            pltpu.VMEM_SHARED((NP, HC), jnp.float32),
            pltpu.VMEM_SHARED((NP, 16), jnp.float32),
        ],
    )
    def k(src_hbm, dst_hbm, asrc_hbm, adst_hbm, xw_hbm, z128_hbm, z16_hbm,
          acc_out, den_out, sidx, didx, gsrc, gdst, exv, xwv, msg,
          acc_sh, den_sh):
        cid = lax.axis_index("c")
        sid = lax.axis_index("s")
        wid = sid * NUM_SC + cid
        r0 = sid * ROWS_PER_SUB

        pltpu.sync_copy(z128_hbm.at[pl.ds(r0, ROWS_PER_SUB), :],
                        acc_sh.at[pl.ds(r0, ROWS_PER_SUB), :])
        pltpu.sync_copy(z16_hbm.at[pl.ds(r0, ROWS_PER_SUB), :],
                        den_sh.at[pl.ds(r0, ROWS_PER_SUB), :])
        plsc.subcore_barrier()

        ebase = wid * EDGES_PER_W

        @pl.loop(0, NCHUNK)
        def _chunk(ci):
            base = ebase + ci * CHUNK
            pltpu.sync_copy(src_hbm.at[pl.ds(base, CHUNK)], sidx.at[0])
            pltpu.sync_copy(dst_hbm.at[pl.ds(base, CHUNK)], didx.at[0])
            pltpu.sync_copy(asrc_hbm.at[sidx.at[0]], gsrc)
            pltpu.sync_copy(adst_hbm.at[didx.at[0]], gdst)
            pltpu.sync_copy(xw_hbm.at[sidx.at[0]], xwv)

            @pl.loop(0, CHUNK)
            def _edge(e):
                v = gsrc[e, :] + gdst[e, :]
                lk = jnp.maximum(v, v * NEG_SLOPE)
                ex = jnp.exp(lk)
                exv[e, :] = ex
                for h in range(H):
                    xr = xwv[e, pl.ds(h * 16, 16)]
                    cb = plsc.load_gather(
                        exv,
                        [jnp.full((16,), e, jnp.int32),
                         jnp.full((16,), h, jnp.int32)],
                    )
                    msg[e, pl.ds(h * 16, 16)] = xr * cb

            pltpu.sync_copy(exv, den_sh.at[didx.at[0]], add=True)
            pltpu.sync_copy(msg, acc_sh.at[didx.at[0]], add=True)

        plsc.subcore_barrier()
        pltpu.sync_copy(acc_sh.at[pl.ds(r0, ROWS_PER_SUB), :],
                        acc_out.at[cid, pl.ds(r0, ROWS_PER_SUB), :])
        pltpu.sync_copy(den_sh.at[pl.ds(r0, ROWS_PER_SUB), :],
                        den_out.at[cid, pl.ds(r0, ROWS_PER_SUB), :])

    return k(src, dst, asrc, adst, xw, zeros128, zeros16)


def _norm_body(a0_ref, a1_ref, d0_ref, d1_ref, b_ref, out_ref):
    acc = a0_ref[...] + a1_ref[...]
    den = d0_ref[...] + d1_ref[...]
    for h in range(H):
        dh = den[:, h:h + 1]
        safe = jnp.where(dh == 0.0, 1.0, dh)
        sl = slice(h * 16, (h + 1) * 16)
        out_ref[:, sl] = acc[:, sl] / safe + b_ref[:, sl]


def _normalize(acc0, acc1, den0, den1, bias2d):
    blk = 640
    grid = (NP // blk,)
    return pl.pallas_call(
        _norm_body,
        grid=grid,
        in_specs=[
            pl.BlockSpec((blk, HC), lambda i: (i, 0)),
            pl.BlockSpec((blk, HC), lambda i: (i, 0)),
            pl.BlockSpec((blk, 16), lambda i: (i, 0)),
            pl.BlockSpec((blk, 16), lambda i: (i, 0)),
            pl.BlockSpec((1, HC), lambda i: (0, 0)),
        ],
        out_specs=pl.BlockSpec((blk, HC), lambda i: (i, 0)),
        out_shape=jax.ShapeDtypeStruct((NP, HC), jnp.float32),
    )(acc0, acc1, den0, den1, bias2d)


def kernel(x, edge_index, W, att_src, att_dst, bias):
    src = edge_index[0]
    dst = edge_index[1]

    eye = jnp.eye(H, dtype=jnp.float32)
    ms = (att_src[0][:, :, None] * eye[:, None, :]).reshape(HC, H)
    md = (att_dst[0][:, :, None] * eye[:, None, :]).reshape(HC, H)
    AB = jnp.concatenate([ms, ms, md, md], axis=1)

    xw, asrc_p, adst_p = _project(x, W, AB)

    zeros128 = jnp.zeros((NP, HC), jnp.float32)
    zeros16 = jnp.zeros((NP, 16), jnp.float32)
    acc_p, den_p = _sc_edge_kernel(src, dst, asrc_p, adst_p, xw,
                                   zeros128, zeros16)

    out = _normalize(acc_p[0], acc_p[1], den_p[0], den_p[1],
                     bias.reshape(1, HC))
    return out[:N]
